# baseline (speedup 1.0000x reference)
import functools

import jax
import jax.numpy as jnp
from jax.experimental import pallas as pl
from jax.experimental.pallas import tpu as pltpu

_BM = 480
_BX = 1008


def _fused_kernel(x_ref, w_ref, adj_ref, b_ref, o_ref, h_ref, *, gx):
    i = pl.program_id(0)

    @pl.when(i < gx)
    def _build_h_slice():
        h = jnp.dot(x_ref[...], w_ref[...],
                    preferred_element_type=jnp.float32)
        h_ref[pl.ds(i * _BX, _BX), :] = h.astype(jnp.bfloat16)

    @pl.when(i >= gx)
    def _aggregate_band():
        n = adj_ref.shape[1]
        acc = jax.lax.dot_general(
            adj_ref[...], h_ref[:n, :],
            dimension_numbers=(((1,), (0,)), ((), ())),
            preferred_element_type=jnp.float32,
        )
        o_ref[...] = acc + b_ref[...]


def kernel(input, adj, W, b):
    n, d_in = input.shape
    d_out = W.shape[1]
    gx = pl.cdiv(n, _BX)
    gm = pl.cdiv(n, _BM)

    body = functools.partial(_fused_kernel, gx=gx)
    last_x = gx - 1

    return pl.pallas_call(
        body,
        grid=(gx + gm,),
        in_specs=[
            pl.BlockSpec((_BX, d_in), lambda i: (jnp.minimum(i, last_x), 0)),
            pl.BlockSpec((d_in, d_out), lambda i: (0, 0)),
            pl.BlockSpec((_BM, n), lambda i: (jnp.maximum(i - gx, 0), 0)),
            pl.BlockSpec((1, d_out), lambda i: (0, 0)),
        ],
        out_specs=pl.BlockSpec(
            (_BM, d_out), lambda i: (jnp.maximum(i - gx, 0), 0)),
        out_shape=jax.ShapeDtypeStruct((n, d_out), jnp.float32),
        scratch_shapes=[
            pltpu.VMEM((gx * _BX, d_out), jnp.bfloat16),
        ],
        compiler_params=pltpu.CompilerParams(
            dimension_semantics=("arbitrary",),
        ),
    )(input, W, adj, b)
